# Initial kernel scaffold; baseline (speedup 1.0000x reference)
import jax
import jax.numpy as jnp
from jax.experimental import pallas as pl


def kernel(x, edge_index, edge_features, params):
    raise NotImplementedError("write your pallas kernel here")


# speedup vs baseline: 1.0000x; 1.0000x over previous
import functools

import jax
import jax.numpy as jnp
from jax.experimental import pallas as pl
from jax.experimental.pallas import tpu as pltpu


def _ln_relu_body(x_ref, g_ref, b_ref, o_ref, *, relu):
    x = x_ref[...]
    mu = jnp.mean(x, axis=-1, keepdims=True)
    var = jnp.mean((x - mu) ** 2, axis=-1, keepdims=True)
    y = (x - mu) * jax.lax.rsqrt(var + 1e-5) * g_ref[...] + b_ref[...]
    if relu:
        y = jnp.maximum(y, 0.0)
    o_ref[...] = y


def _ln_relu(x, g, b, relu):
    n, d = x.shape
    blk = 1000
    return pl.pallas_call(
        functools.partial(_ln_relu_body, relu=relu),
        grid=(n // blk,),
        in_specs=[
            pl.BlockSpec((blk, d), lambda i: (i, 0)),
            pl.BlockSpec((d,), lambda i: (0,)),
            pl.BlockSpec((d,), lambda i: (0,)),
        ],
        out_specs=pl.BlockSpec((blk, d), lambda i: (i, 0)),
        out_shape=jax.ShapeDtypeStruct((n, d), x.dtype),
    )(x, g, b)


def _gatv2_layer(x, src, dst, ea, p):
    n = x.shape[0]
    xl = x @ p['Wl'] + p['bl']
    xr = x @ p['Wr'] + p['br']
    ep = ea @ p['We']
    loop = jnp.arange(n, dtype=src.dtype)
    cnt = jax.ops.segment_sum(jnp.ones((src.shape[0],), dtype=x.dtype), dst, num_segments=n)
    loop_ea = jax.ops.segment_sum(ea, dst, num_segments=n) / jnp.clip(cnt, 1.0)[:, None]
    ep_a = jnp.concatenate([ep, loop_ea @ p['We']], axis=0)
    src_a = jnp.concatenate([src, loop])
    dst_a = jnp.concatenate([dst, loop])
    feat = xl[src_a] + xr[dst_a] + ep_a
    feat = jnp.where(feat > 0, feat, 0.2 * feat)
    e = feat @ p['att']
    m = jax.ops.segment_max(e, dst_a, num_segments=n)
    ex = jnp.exp(e - m[dst_a])
    den = jax.ops.segment_sum(ex, dst_a, num_segments=n)
    alpha = ex / (den[dst_a] + 1e-16)
    out = jax.ops.segment_sum(alpha[:, None] * xl[src_a], dst_a, num_segments=n)
    return out + p['bias']


def kernel(x, edge_index, edge_features, params):
    src, dst = edge_index[0], edge_index[1]
    out = _gatv2_layer(x, src, dst, edge_features, params['conv1'])
    out = _ln_relu(out, params['conv1']['g'], params['conv1']['b'], True)
    out = _gatv2_layer(out, src, dst, edge_features, params['conv2'])
    out = _ln_relu(out, params['conv2']['g'], params['conv2']['b'], True)
    out = _gatv2_layer(out, src, dst, edge_features, params['conv3'])
    out = _ln_relu(out, params['conv3']['g'], params['conv3']['b'], False)
    return out


# speedup vs baseline: 1.3786x; 1.3786x over previous
import functools

import jax
import jax.numpy as jnp
from jax import lax
from jax.experimental import pallas as pl
from jax.experimental.pallas import tpu as pltpu
from jax.experimental.pallas import tpu_sc as plsc

_NC = 2
_NS = 16
_NW = _NC * _NS
_LANES = 16


def _sc_mesh():
    return plsc.VectorSubcoreMesh(
        core_axis_name="c", subcore_axis_name="s", num_cores=_NC, num_subcores=_NS
    )


def _edge_scores_sc(xl, xr, ep, src, dst, att):
    e_total, dout = ep.shape
    n = xl.shape[0]
    per_w = e_total // _NW
    blk = 80
    nblk = per_w // blk
    nch = dout // _LANES
    ngrp = blk // _LANES

    @functools.partial(
        pl.kernel,
        out_type=jax.ShapeDtypeStruct((e_total,), jnp.float32),
        mesh=_sc_mesh(),
        compiler_params=pltpu.CompilerParams(
            use_tc_tiling_on_sc=False, needs_layout_passes=False),
        scratch_types=[
            pltpu.VMEM((blk,), jnp.int32),
            pltpu.VMEM((blk,), jnp.int32),
            pltpu.VMEM((blk, dout), jnp.float32),
            pltpu.VMEM((blk, dout), jnp.float32),
            pltpu.VMEM((blk, dout), jnp.float32),
            pltpu.VMEM((dout,), jnp.float32),
            pltpu.VMEM((_LANES * _LANES,), jnp.float32),
            pltpu.VMEM((blk,), jnp.float32),
            pltpu.SemaphoreType.DMA,
            pltpu.SemaphoreType.DMA,
        ],
    )
    def kern(xl_h, xr_h, ep_h, src_h, dst_h, att_h, e_h,
             si, di, rl, rr, re, attv, dots, ebuf, sem1, sem2):
        wid = lax.axis_index("s") * _NC + lax.axis_index("c")
        pltpu.sync_copy(att_h, attv)
        lane = lax.iota(jnp.int32, _LANES)

        def blk_body(b, _):
            base = wid * per_w + b * blk
            pltpu.sync_copy(src_h.at[pl.ds(base, blk)], si)
            pltpu.sync_copy(dst_h.at[pl.ds(base, blk)], di)
            cl = pltpu.async_copy(xl_h.at[si], rl, sem1)
            cr = pltpu.async_copy(xr_h.at[di], rr, sem2)
            pltpu.sync_copy(ep_h.at[pl.ds(base, blk)], re)
            cl.wait()
            cr.wait()

            def grp_body(g, _):
                for j in range(_LANES):
                    row = jnp.full((_LANES,), g * _LANES + j, jnp.int32)
                    acc = jnp.zeros((_LANES,), jnp.float32)
                    for c in range(nch):
                        col = c * _LANES + lane
                        s = (plsc.load_gather(rl, [row, col])
                             + plsc.load_gather(rr, [row, col])
                             + plsc.load_gather(re, [row, col]))
                        s = jnp.maximum(s, 0.2 * s)
                        acc = acc + s * attv[pl.ds(c * _LANES, _LANES)]
                    dots[pl.ds(j * _LANES, _LANES)] = acc
                e16 = jnp.zeros((_LANES,), jnp.float32)
                for t in range(_LANES):
                    e16 = e16 + plsc.load_gather(dots, [lane * _LANES + t])
                ebuf[pl.ds(g * _LANES, _LANES)] = e16
                return 0

            lax.fori_loop(0, ngrp, grp_body, 0)
            pltpu.sync_copy(ebuf, e_h.at[pl.ds(base, blk)])
            return 0

        lax.fori_loop(0, nblk, blk_body, 0)

    return kern(xl, xr, ep, src, dst, att)


def _gatv2_layer(x, src, dst, ea, p):
    n = x.shape[0]
    xl = x @ p['Wl'] + p['bl']
    xr = x @ p['Wr'] + p['br']
    ep = ea @ p['We']
    loop = jnp.arange(n, dtype=src.dtype)
    cnt = jax.ops.segment_sum(jnp.ones((src.shape[0],), dtype=x.dtype), dst, num_segments=n)
    loop_ea = jax.ops.segment_sum(ea, dst, num_segments=n) / jnp.clip(cnt, 1.0)[:, None]
    lep = loop_ea @ p['We']

    e_edge = _edge_scores_sc(xl, xr, ep, src, dst, p['att'])
    sf = xl + xr + lep
    sf = jnp.where(sf > 0, sf, 0.2 * sf)
    e_self = sf @ p['att']

    m_e = jax.ops.segment_max(e_edge, dst, num_segments=n)
    m_t = jnp.maximum(m_e, e_self)
    ex = jnp.exp(e_edge - m_t[dst])
    ex_self = jnp.exp(e_self - m_t)
    den = jax.ops.segment_sum(ex, dst, num_segments=n) + ex_self
    num = jax.ops.segment_sum(ex[:, None] * xl[src], dst, num_segments=n) + ex_self[:, None] * xl
    out = num / (den[:, None] + 1e-16)
    return out + p['bias']


def _ln_relu_body(x_ref, g_ref, b_ref, o_ref, *, relu):
    x = x_ref[...]
    mu = jnp.mean(x, axis=-1, keepdims=True)
    var = jnp.mean((x - mu) ** 2, axis=-1, keepdims=True)
    y = (x - mu) * jax.lax.rsqrt(var + 1e-5) * g_ref[...] + b_ref[...]
    if relu:
        y = jnp.maximum(y, 0.0)
    o_ref[...] = y


def _ln_relu(x, g, b, relu):
    n, d = x.shape
    blk = 1000
    return pl.pallas_call(
        functools.partial(_ln_relu_body, relu=relu),
        grid=(n // blk,),
        in_specs=[
            pl.BlockSpec((blk, d), lambda i: (i, 0)),
            pl.BlockSpec((d,), lambda i: (0,)),
            pl.BlockSpec((d,), lambda i: (0,)),
        ],
        out_specs=pl.BlockSpec((blk, d), lambda i: (i, 0)),
        out_shape=jax.ShapeDtypeStruct((n, d), x.dtype),
    )(x, g, b)


def kernel(x, edge_index, edge_features, params):
    src, dst = edge_index[0], edge_index[1]
    out = _gatv2_layer(x, src, dst, edge_features, params['conv1'])
    out = _ln_relu(out, params['conv1']['g'], params['conv1']['b'], True)
    out = _gatv2_layer(out, src, dst, edge_features, params['conv2'])
    out = _ln_relu(out, params['conv2']['g'], params['conv2']['b'], True)
    out = _gatv2_layer(out, src, dst, edge_features, params['conv3'])
    out = _ln_relu(out, params['conv3']['g'], params['conv3']['b'], False)
    return out
